# baseline (speedup 1.0000x reference)
import functools

import jax
import jax.numpy as jnp
from jax import lax
from jax.experimental import pallas as pl
from jax.experimental.pallas import tpu as pltpu
from jax.experimental.pallas import tpu_sc as plsc

_D = 64
_NC = 2
_NS = 16
_NW = _NC * _NS
_SUBI = 128


@functools.lru_cache(maxsize=None)
def _make_lookup(NB: int, NH: int, V: int):
    bpw = NB // _NW
    nbt = bpw // _SUBI
    slab = _D * bpw
    assert NB % (_NW * _SUBI) == 0 and NH % 2 == 0
    mesh = plsc.VectorSubcoreMesh(core_axis_name="c", subcore_axis_name="s")

    @functools.partial(
        pl.kernel,
        out_type=jax.ShapeDtypeStruct((NH * _D * NB,), jnp.float32),
        mesh=mesh,
        compiler_params=pltpu.CompilerParams(
            needs_layout_passes=False, use_tc_tiling_on_sc=False),
        scratch_types=[
            pltpu.VMEM((NH, bpw), jnp.int32),
            pltpu.VMEM((2, bpw, _D), jnp.int8),
            pltpu.VMEM((2, slab), jnp.float32),
            pltpu.VMEM((16,), jnp.float32),
            pltpu.SemaphoreType.DMA,
            pltpu.SemaphoreType.DMA,
            pltpu.SemaphoreType.DMA,
            pltpu.SemaphoreType.DMA,
        ],
    )
    def lookup(idx_hbm, tbl_hbm, inv_hbm, out_hbm,
               idx_v, raw_v, slab_v, inv_v, gsem0, gsem1, osem0, osem1):
        wid = lax.axis_index("s") * _NC + lax.axis_index("c")
        b0 = wid * bpw
        gsem = (gsem0, gsem1)
        osem = (osem0, osem1)
        pltpu.sync_copy(inv_hbm, inv_v)
        pltpu.sync_copy(idx_hbm.at[:, pl.ds(b0, bpw)], idx_v)
        inv = inv_v[...]
        iot = lax.iota(jnp.int32, 16)
        pat = (iot >> 1) * (nbt * 1024) + (iot & 1) * 512

        def issue_gather(h, b):
            for s in range(nbt):
                pltpu.make_async_copy(
                    tbl_hbm.at[idx_v.at[h, pl.ds(s * _SUBI, _SUBI)]],
                    raw_v.at[b, pl.ds(s * _SUBI, _SUBI)],
                    gsem[b]).start()

        def wait_gather(b):
            pltpu.make_async_copy(
                tbl_hbm.at[pl.ds(0, bpw)], raw_v.at[b], gsem[b]).wait()

        def issue_out(h, b):
            piece = nbt * 1024
            for dq in range(8):
                pltpu.make_async_copy(
                    slab_v.at[b, pl.ds(dq * piece, piece)],
                    out_hbm.at[pl.ds(
                        h * (_D * NB) + dq * (8 * NB) + wid * piece,
                        piece)],
                    osem[b]).start()

        def wait_out(b):
            pltpu.make_async_copy(
                slab_v.at[b], out_hbm.at[pl.ds(0, slab)], osem[b]).wait()

        def compute(b):
            def row_body(r, c2):
                words = plsc.bitcast(raw_v[b, r], jnp.int32)
                base = (r >> 7) * 1024 + (r & 127)
                for k in range(4):
                    if k < 3:
                        byte = (words << (24 - 8 * k)) >> 24
                    else:
                        byte = words >> 24
                    val = byte.astype(jnp.float32) * inv
                    plsc.store_scatter(
                        slab_v.at[b], [pat + (base + k * 128)], val)
                return c2
            lax.fori_loop(0, bpw, row_body, 0)

        issue_gather(0, 0)

        def pair_body(t, carry):
            for b in range(2):
                h = 2 * t + b

                @pl.when(h + 1 < NH)
                def _():
                    issue_gather(h + 1, 1 - b)

                wait_gather(b)

                @pl.when(h >= 2)
                def _():
                    wait_out(b)

                compute(b)
                issue_out(h, b)
            return carry

        lax.fori_loop(0, NH // 2, pair_body, 0)
        wait_out(0)
        wait_out(1)

    return lookup


@jax.jit
def kernel(x, weight_quant, c):
    batch, hist = x.shape
    xt = x.astype(jnp.int32).T
    inv = jnp.broadcast_to((1.0 / c).astype(jnp.float32), (16,))
    v = weight_quant.shape[0]
    lookup = _make_lookup(batch, hist, v)
    out1d = lookup(xt, weight_quant, inv)
    o5 = out1d.reshape(hist, 8, batch // 128, 8, 128)
    return o5.transpose(2, 4, 0, 1, 3).reshape(batch, hist, _D)
